# baseline (speedup 1.0000x reference)
import functools

import numpy as np
import jax
import jax.numpy as jnp
from jax import lax
from jax.experimental import pallas as pl
from jax.experimental.pallas import tpu as pltpu
from jax.experimental.pallas import tpu_sc as plsc

_H = 768
_NT = 18
_OROW = 128
_SCOL = 18
_W = _H + 128
_MAX_SPAN = 8
_B, _S = 4, 512
_NSP = 4068
_SEG = _S * _MAX_SPAN
_R = 256
_TROWS = 2304
_NTILES = 32
_TSTARTS = 64
_SUB = 16
_NSUBS = _TSTARTS // _SUB
_BLK = _SUB * _MAX_SPAN
_TOT = _B * _SEG
_HB_UNROLL = 3


def _dest_indices():
    dest = np.zeros((_B, _S, _MAX_SPAN), np.int64)
    for b in range(_B):
        pos = b * _SEG
        dump = b * _SEG + _NSP
        for s in range(_S):
            for k in range(_MAX_SPAN):
                if s + k < _S:
                    dest[b, s, k] = pos
                    pos += 1
                else:
                    dest[b, s, k] = dump
                    dump += 1
    didx = np.zeros(_TOT, np.int64)
    p = 0
    for wid in range(_NTILES):
        b, tb = wid // 8, wid % 8
        for s_local in range(_TSTARTS):
            s = tb * _TSTARTS + s_local
            for k in range(_MAX_SPAN):
                didx[p] = dest[b, s, k]
                p += 1
    return didx.astype(np.int32)


_DIDX_NP = _dest_indices()


def _proj_body(hs_ref, ws1_ref, we_ref, wb_ref, ws2_ref,
               bs1_ref, be_ref, bbd_ref, bs2_ref,
               ts_ref, te_ref, bnd_ref, wpar_ref):
    hs = hs_ref[0]
    w1 = ws1_ref[...]
    we = we_ref[...]
    be2 = 0.5 * be_ref[...]
    dn = (((1,), (1,)), ((), ()))
    dot = lambda a, b: lax.dot_general(a, b, dn,
                                       preferred_element_type=jnp.float32)
    ts_ref[:, : _H] = dot(hs, w1[:, :_H]) + bs1_ref[...]
    ts_ref[:, _H : _H + _NT] = dot(hs, we[:, :_H]) + be2
    te_ref[:, : _H] = dot(hs, w1[:, _H:])
    te_ref[:, _H : _H + _NT] = dot(hs, we[:, _H:]) + be2
    bnd_ref[0] = dot(hs, wb_ref[...]) + bbd_ref[...]
    wpar_ref[:, : _H] = ws2_ref[...]
    wpar_ref[:, _H:] = jnp.broadcast_to(bs2_ref[...], (1, 16))


def _proj_call(hidden_states, ws1, we, wb, ws2, bs1, be, bbd, bs2):
    grid = (_B, _S // _R)
    full = lambda shape: pl.BlockSpec(shape, lambda b, j: (0,) * len(shape))
    return pl.pallas_call(
        _proj_body,
        grid=grid,
        in_specs=[
            pl.BlockSpec((1, _R, _H), lambda b, j: (b, j, 0)),
            full((_H, 2 * _H)), full((_NT, 2 * _H)), full((3, _H)),
            full((1, _H)),
            full((1, _H)), full((1, _NT)), full((1, 3)), full((1, 1)),
        ],
        out_specs=[
            pl.BlockSpec((_R, _W), lambda b, j: (2 * b + j, 0)),
            pl.BlockSpec((_R, _W), lambda b, j: (2 * b + j, 0)),
            pl.BlockSpec((1, _R, 3), lambda b, j: (b, j, 0)),
            pl.BlockSpec((1, _H + 16), lambda b, j: (0, 0)),
        ],
        out_shape=[
            jax.ShapeDtypeStruct((_TROWS, _W), jnp.float32),
            jax.ShapeDtypeStruct((_TROWS, _W), jnp.float32),
            jax.ShapeDtypeStruct((_B, _S, 3), jnp.float32),
            jax.ShapeDtypeStruct((1, _H + 16), jnp.float32),
        ],
    )(hidden_states, ws1, we, wb, ws2, bs1, be, bbd, bs2)


@functools.lru_cache(maxsize=1)
def _span_call():
    mesh = plsc.VectorSubcoreMesh(core_axis_name="c", subcore_axis_name="s",
                                  num_cores=2, num_subcores=16)

    @functools.partial(
        pl.kernel,
        out_type=jax.ShapeDtypeStruct((_TOT, _OROW), jnp.float32),
        mesh=mesh,
        compiler_params=pltpu.CompilerParams(needs_layout_passes=False),
        scratch_types=[
            pltpu.VMEM((2, _SUB, _W), jnp.float32),
            pltpu.VMEM((2, _SUB + _MAX_SPAN, _W), jnp.float32),
            pltpu.VMEM((2, _BLK, _OROW), jnp.float32),
            pltpu.VMEM((2, _BLK), jnp.int32),
            pltpu.VMEM((_H,), jnp.float32),
            pltpu.VMEM((16,), jnp.float32),
            pltpu.SemaphoreType.DMA,
            pltpu.SemaphoreType.DMA,
            pltpu.SemaphoreType.DMA,
            pltpu.SemaphoreType.DMA,
        ],
    )
    def span_kernel(ts_hbm, te_hbm, didx_hbm, wpar_hbm, out_hbm,
                    a_v, b_v, o_v, didx_v, w_v, bi_v,
                    sema, semb, semd, semo):
        wid = lax.axis_index("s") * 2 + lax.axis_index("c")
        pltpu.sync_copy(wpar_hbm.at[0, pl.ds(0, _H)], w_v)
        pltpu.sync_copy(wpar_hbm.at[0, pl.ds(_H, 16)], bi_v)
        row0 = (wid // 8) * _S + (wid % 8) * _TSTARTS
        obase0 = wid * _TSTARTS * _MAX_SPAN
        lane = lax.iota(jnp.int32, 16)
        zero16 = jnp.zeros((16,), jnp.float32)

        def issue_in(sub):
            slot = sub % 2
            r0 = row0 + sub * _SUB
            obase = obase0 + sub * _BLK
            ca = pltpu.async_copy(ts_hbm.at[pl.ds(r0, _SUB)],
                                  a_v.at[slot], sema)
            cb = pltpu.async_copy(te_hbm.at[pl.ds(r0, _SUB + _MAX_SPAN)],
                                  b_v.at[slot], semb)
            cd = pltpu.async_copy(didx_hbm.at[pl.ds(obase, _BLK)],
                                  didx_v.at[slot], semd)
            return ca, cb, cd

        pend_in = issue_in(0)
        pend_out = [None, None]
        for sub in range(_NSUBS):
            slot = sub % 2
            for c in pend_in:
                c.wait()
            if sub + 1 < _NSUBS:
                nslot = (sub + 1) % 2
                if pend_out[nslot] is not None:
                    pend_out[nslot].wait()
                    pend_out[nslot] = None
                pend_in = issue_in(sub + 1)
            a_s = a_v.at[slot]
            b_s = b_v.at[slot]
            o_s = o_v.at[slot]
            if pend_out[slot] is not None:
                pend_out[slot].wait()
                pend_out[slot] = None

            def start_body(i, carry2, a_s=a_s, b_s=b_s, o_s=o_s):
                def hb_body(t, accs):
                    for u in range(_HB_UNROLL):
                        hb = t * _HB_UNROLL + u
                        sl = pl.ds(hb * 16, 16)
                        va = a_s[i, sl]
                        vw = w_v[sl]
                        accs = tuple(
                            accs[k] + jnp.maximum(va + b_s[i + k, sl], 0.0)
                            * vw
                            for k in range(_MAX_SPAN)
                        )
                    return accs

                accs = lax.fori_loop(0, _H // 16 // _HB_UNROLL, hb_body,
                                     (zero16,) * _MAX_SPAN)
                lo, hi = pl.ds(_H, 16), pl.ds(_H + 16, 16)
                ealo = a_s[i, lo]
                eahi = a_s[i, hi]
                for k in range(_MAX_SPAN):
                    x = accs[k]
                    for sh in (8, 4, 2, 1):
                        x = x + jnp.take_along_axis(
                            x, lane ^ sh, axis=0, mode="promise_in_bounds")
                    x = x + bi_v[...]
                    r = i * _MAX_SPAN + k
                    o_s[r, pl.ds(0, 16)] = ealo + b_s[i + k, lo]
                    o_s[r, pl.ds(16, 16)] = jnp.where(
                        lane == _SCOL - 16, x, eahi + b_s[i + k, hi])
                return carry2

            lax.fori_loop(0, _SUB, start_body, 0)
            pend_out[slot] = pltpu.async_copy(
                o_s, out_hbm.at[didx_v.at[slot]], semo)
        for c in pend_out:
            if c is not None:
                c.wait()

    return span_kernel


def _extract_body(in_ref, sc_ref, ent_ref):
    blk = in_ref[0]
    ent_ref[0] = blk[: _NSP, : _NT]
    sc_ref[0] = blk[: _NSP, _SCOL : _SCOL + 1]


def _extract_call(packed):
    return pl.pallas_call(
        _extract_body,
        grid=(_B,),
        in_specs=[pl.BlockSpec((1, _SEG, _OROW), lambda b: (b, 0, 0))],
        out_specs=[
            pl.BlockSpec((1, _NSP, 1), lambda b: (b, 0, 0)),
            pl.BlockSpec((1, _NSP, _NT), lambda b: (b, 0, 0)),
        ],
        out_shape=[
            jax.ShapeDtypeStruct((_B, _NSP, 1), jnp.float32),
            jax.ShapeDtypeStruct((_B, _NSP, _NT), jnp.float32),
        ],
    )(packed)


def kernel(hidden_states, attention_mask, W_b, b_b, W_e, b_e,
           W_s1, b_s1, W_s2, b_s2):
    del attention_mask

    ts, te, bnd, wpar = _proj_call(
        hidden_states, W_s1, W_e, W_b, W_s2,
        b_s1.reshape(1, _H), b_e.reshape(1, _NT), b_b.reshape(1, 3),
        b_s2.reshape(1, 1))

    out = _span_call()(ts, te, jnp.asarray(_DIDX_NP), wpar)

    span_scores, entity_logits = _extract_call(
        out.reshape(_B, _SEG, _OROW))
    return bnd, span_scores, entity_logits


# speedup vs baseline: 1.1859x; 1.1859x over previous
import functools

import numpy as np
import jax
import jax.numpy as jnp
from jax import lax
from jax.experimental import pallas as pl
from jax.experimental.pallas import tpu as pltpu
from jax.experimental.pallas import tpu_sc as plsc

_H = 768
_NT = 18
_NTP = 24
_W = _H + 128
_MAX_SPAN = 8
_B, _S = 4, 512
_NSP = 4068
_SEG = _S * _MAX_SPAN
_R = 256
_TROWS = 2304
_NTILES = 32
_TSTARTS = 64
_SUB = 16
_NSUBS = _TSTARTS // _SUB
_BLK = _SUB * _MAX_SPAN
_ESTR = 129
_HB_UNROLL = 3


def _proj_body(hs_ref, ws1_ref, we_ref, wb_ref, ws2_ref,
               bs1_ref, be_ref, bbd_ref, bs2_ref,
               ts_ref, te_ref, bnd_ref, wpar_ref):
    hs = hs_ref[0]
    w1 = ws1_ref[...]
    we = we_ref[...]
    be2 = 0.5 * be_ref[...]
    dn = (((1,), (1,)), ((), ()))
    dot = lambda a, b: lax.dot_general(a, b, dn,
                                       preferred_element_type=jnp.float32)
    ts_ref[:, : _H] = dot(hs, w1[:, :_H]) + bs1_ref[...]
    ts_ref[:, _H : _H + _NT] = dot(hs, we[:, :_H]) + be2
    te_ref[:, : _H] = dot(hs, w1[:, _H:])
    te_ref[:, _H : _H + _NT] = dot(hs, we[:, _H:]) + be2
    bnd_ref[0] = dot(hs, wb_ref[...]) + bbd_ref[...]
    wpar_ref[:, : _H] = ws2_ref[...]
    wpar_ref[:, _H:] = jnp.broadcast_to(bs2_ref[...], (1, 16))


def _proj_call(hidden_states, ws1, we, wb, ws2, bs1, be, bbd, bs2):
    grid = (_B, _S // _R)
    full = lambda shape: pl.BlockSpec(shape, lambda b, j: (0,) * len(shape))
    return pl.pallas_call(
        _proj_body,
        grid=grid,
        in_specs=[
            pl.BlockSpec((1, _R, _H), lambda b, j: (b, j, 0)),
            full((_H, 2 * _H)), full((_NT, 2 * _H)), full((3, _H)),
            full((1, _H)),
            full((1, _H)), full((1, _NT)), full((1, 3)), full((1, 1)),
        ],
        out_specs=[
            pl.BlockSpec((_R, _W), lambda b, j: (2 * b + j, 0)),
            pl.BlockSpec((_R, _W), lambda b, j: (2 * b + j, 0)),
            pl.BlockSpec((1, _R, 3), lambda b, j: (b, j, 0)),
            pl.BlockSpec((1, _H + 16), lambda b, j: (0, 0)),
        ],
        out_shape=[
            jax.ShapeDtypeStruct((_TROWS, _W), jnp.float32),
            jax.ShapeDtypeStruct((_TROWS, _W), jnp.float32),
            jax.ShapeDtypeStruct((_B, _S, 3), jnp.float32),
            jax.ShapeDtypeStruct((1, _H + 16), jnp.float32),
        ],
    )(hidden_states, ws1, we, wb, ws2, bs1, be, bbd, bs2)


@functools.lru_cache(maxsize=1)
def _span_call():
    mesh = plsc.VectorSubcoreMesh(core_axis_name="c", subcore_axis_name="s",
                                  num_cores=2, num_subcores=16)

    @functools.partial(
        pl.kernel,
        out_type=[
            jax.ShapeDtypeStruct((_B * _SEG,), jnp.float32),
            jax.ShapeDtypeStruct((_B, _NTP, _SEG), jnp.float32),
        ],
        mesh=mesh,
        compiler_params=pltpu.CompilerParams(needs_layout_passes=False),
        scratch_types=[
            pltpu.VMEM((2, _SUB, _W), jnp.float32),
            pltpu.VMEM((2, _SUB + _MAX_SPAN, _W), jnp.float32),
            pltpu.VMEM((2, 32, _ESTR), jnp.float32),
            pltpu.VMEM((2, _BLK), jnp.float32),
            pltpu.VMEM((_H,), jnp.float32),
            pltpu.VMEM((16,), jnp.float32),
            pltpu.SemaphoreType.DMA,
            pltpu.SemaphoreType.DMA,
            pltpu.SemaphoreType.DMA,
        ],
    )
    def span_kernel(ts_hbm, te_hbm, wpar_hbm, osc_hbm, oent_hbm,
                    a_v, b_v, et_v, sc_v, w_v, bi_v,
                    sema, semb, semo):
        wid = lax.axis_index("s") * 2 + lax.axis_index("c")
        pltpu.sync_copy(wpar_hbm.at[0, pl.ds(0, _H)], w_v)
        pltpu.sync_copy(wpar_hbm.at[0, pl.ds(_H, 16)], bi_v)
        bb = wid // 8
        s0t = (wid % 8) * _TSTARTS
        row0 = bb * _S + s0t
        lane = lax.iota(jnp.int32, 16)
        lane7 = jnp.minimum(lane, 7)
        zero16 = jnp.zeros((16,), jnp.float32)

        def issue_in(sub):
            slot = sub % 2
            r0 = row0 + sub * _SUB
            ca = pltpu.async_copy(ts_hbm.at[pl.ds(r0, _SUB)],
                                  a_v.at[slot], sema)
            cb = pltpu.async_copy(te_hbm.at[pl.ds(r0, _SUB + _MAX_SPAN)],
                                  b_v.at[slot], semb)
            return ca, cb

        pend_in = issue_in(0)
        pend_out = [(), ()]
        for sub in range(_NSUBS):
            slot = sub % 2
            for c in pend_in:
                c.wait()
            if sub + 1 < _NSUBS:
                pend_in = issue_in(sub + 1)
            a_s = a_v.at[slot]
            b_s = b_v.at[slot]
            e_s = et_v.at[slot]
            c_s = sc_v.at[slot]
            for c in pend_out[slot]:
                c.wait()
            pend_out[slot] = ()
            s0b = s0t + sub * _SUB

            def start_body(i, carry2, a_s=a_s, b_s=b_s, e_s=e_s, c_s=c_s,
                           s0b=s0b):
                def hb_body(t, accs):
                    for u in range(_HB_UNROLL):
                        hb = t * _HB_UNROLL + u
                        sl = pl.ds(hb * 16, 16)
                        va = a_s[i, sl]
                        vw = w_v[sl]
                        accs = tuple(
                            accs[k] + jnp.maximum(va + b_s[i + k, sl], 0.0)
                            * vw
                            for k in range(_MAX_SPAN)
                        )
                    return accs

                accs = lax.fori_loop(0, _H // 16 // _HB_UNROLL, hb_body,
                                     (zero16,) * _MAX_SPAN)
                s = s0b + i
                d = jnp.maximum(s - 505, 0)
                lbase = 8 * i - (d * (d + 1)) // 2
                lo, hi = pl.ds(_H, 16), pl.ds(_H + 16, 16)
                ealo = a_s[i, lo]
                eahi = a_s[i, hi]
                xs = []
                for k in range(_MAX_SPAN):
                    x = accs[k]
                    for sh in (8, 4, 2, 1):
                        x = x + jnp.take_along_axis(
                            x, lane ^ sh, axis=0, mode="promise_in_bounds")
                    xs.append(x + bi_v[...])
                    lrow = jnp.full((16,), lbase + k, jnp.int32)
                    plsc.store_scatter(e_s, [lane, lrow],
                                       ealo + b_s[i + k, lo])
                    plsc.store_scatter(e_s, [lane + 16, lrow],
                                       eahi + b_s[i + k, hi])
                svec = xs[7]
                for k in range(7):
                    svec = jnp.where(lane == k, xs[k], svec)
                plsc.store_scatter(c_s, [lbase + lane7], svec)
                return carry2

            lax.fori_loop(0, _SUB, start_body, 0)
            cbase = 8 * s0b
            co1 = pltpu.async_copy(
                c_s, osc_hbm.at[pl.ds(bb * _SEG + cbase, _BLK)], semo)
            co2 = pltpu.async_copy(
                et_v.at[slot, pl.ds(0, _NTP), pl.ds(0, _BLK)],
                oent_hbm.at[bb, :, pl.ds(cbase, _BLK)], semo)
            pend_out[slot] = (co1, co2)
        for pend in pend_out:
            for c in pend:
                c.wait()

    return span_kernel


def kernel(hidden_states, attention_mask, W_b, b_b, W_e, b_e,
           W_s1, b_s1, W_s2, b_s2):
    del attention_mask

    ts, te, bnd, wpar = _proj_call(
        hidden_states, W_s1, W_e, W_b, W_s2,
        b_s1.reshape(1, _H), b_e.reshape(1, _NT), b_b.reshape(1, 3),
        b_s2.reshape(1, 1))

    osc, oent = _span_call()(ts, te, wpar)

    span_scores = osc.reshape(_B, _SEG)[:, :_NSP].reshape(_B, _NSP, 1)
    entity_logits = oent[:, :_NT, :_NSP].transpose(0, 2, 1)
    return bnd, span_scores, entity_logits


# speedup vs baseline: 1.2377x; 1.0437x over previous
import functools

import numpy as np
import jax
import jax.numpy as jnp
from jax import lax
from jax.experimental import pallas as pl
from jax.experimental.pallas import tpu as pltpu
from jax.experimental.pallas import tpu_sc as plsc

_H = 768
_NT = 18
_NTP = 24
_W = _H + 128
_MAX_SPAN = 8
_B, _S = 4, 512
_NSP = 4068
_SEG = _S * _MAX_SPAN
_R = 512
_TROWS = 2304
_NTILES = 32
_TSTARTS = 64
_SUB = 16
_NSUBS = _TSTARTS // _SUB
_BLK = _SUB * _MAX_SPAN
_ESTR = 129
_HB_UNROLL = 2


def _proj_body(hs_ref, ws1_ref, we_ref, wb_ref, ws2_ref,
               bs1_ref, be_ref, bbd_ref, bs2_ref,
               ts_ref, te_ref, bnd_ref, wpar_ref):
    hs = hs_ref[0]
    w1 = ws1_ref[...]
    we = we_ref[...]
    be2 = 0.5 * be_ref[...]
    dn = (((1,), (1,)), ((), ()))
    dot = lambda a, b: lax.dot_general(a, b, dn,
                                       preferred_element_type=jnp.float32)
    ts_ref[:, : _H] = dot(hs, w1[:, :_H]) + bs1_ref[...]
    ts_ref[:, _H : _H + _NT] = dot(hs, we[:, :_H]) + be2
    te_ref[:, : _H] = dot(hs, w1[:, _H:])
    te_ref[:, _H : _H + _NT] = dot(hs, we[:, _H:]) + be2
    bnd_ref[0] = dot(hs, wb_ref[...]) + bbd_ref[...]
    wpar_ref[:, : _H] = ws2_ref[...]
    wpar_ref[:, _H:] = jnp.broadcast_to(bs2_ref[...], (1, 16))


def _proj_call(hidden_states, ws1, we, wb, ws2, bs1, be, bbd, bs2):
    grid = (_B, _S // _R)
    full = lambda shape: pl.BlockSpec(shape, lambda b, j: (0,) * len(shape))
    return pl.pallas_call(
        _proj_body,
        grid=grid,
        in_specs=[
            pl.BlockSpec((1, _R, _H), lambda b, j: (b, j, 0)),
            full((_H, 2 * _H)), full((_NT, 2 * _H)), full((3, _H)),
            full((1, _H)),
            full((1, _H)), full((1, _NT)), full((1, 3)), full((1, 1)),
        ],
        out_specs=[
            pl.BlockSpec((_R, _W), lambda b, j: (b, 0)),
            pl.BlockSpec((_R, _W), lambda b, j: (b, 0)),
            pl.BlockSpec((1, _R, 3), lambda b, j: (b, j, 0)),
            pl.BlockSpec((1, _H + 16), lambda b, j: (0, 0)),
        ],
        out_shape=[
            jax.ShapeDtypeStruct((_TROWS, _W), jnp.float32),
            jax.ShapeDtypeStruct((_TROWS, _W), jnp.float32),
            jax.ShapeDtypeStruct((_B, _S, 3), jnp.float32),
            jax.ShapeDtypeStruct((1, _H + 16), jnp.float32),
        ],
    )(hidden_states, ws1, we, wb, ws2, bs1, be, bbd, bs2)


@functools.lru_cache(maxsize=1)
def _span_call():
    mesh = plsc.VectorSubcoreMesh(core_axis_name="c", subcore_axis_name="s",
                                  num_cores=2, num_subcores=16)

    @functools.partial(
        pl.kernel,
        out_type=[
            jax.ShapeDtypeStruct((_B * _SEG,), jnp.float32),
            jax.ShapeDtypeStruct((_B, _NTP, _SEG), jnp.float32),
        ],
        mesh=mesh,
        compiler_params=pltpu.CompilerParams(needs_layout_passes=False),
        scratch_types=[
            pltpu.VMEM((2, _SUB, _W), jnp.float32),
            pltpu.VMEM((2, _SUB + _MAX_SPAN, _W), jnp.float32),
            pltpu.VMEM((2, 32, _ESTR), jnp.float32),
            pltpu.VMEM((2, _BLK), jnp.float32),
            pltpu.VMEM((_H,), jnp.float32),
            pltpu.VMEM((16,), jnp.float32),
            pltpu.SemaphoreType.DMA,
            pltpu.SemaphoreType.DMA,
            pltpu.SemaphoreType.DMA,
        ],
    )
    def span_kernel(ts_hbm, te_hbm, wpar_hbm, osc_hbm, oent_hbm,
                    a_v, b_v, et_v, sc_v, w_v, bi_v,
                    sema, semb, semo):
        wid = lax.axis_index("s") * 2 + lax.axis_index("c")
        bb = wid // 8
        s0t = (wid % 8) * _TSTARTS
        row0 = bb * _S + s0t
        lane = lax.iota(jnp.int32, 16)
        lane7 = jnp.minimum(lane, 7)
        zero16 = jnp.zeros((16,), jnp.float32)

        def issue_in(sub):
            slot = sub % 2
            r0 = row0 + sub * _SUB
            ca = pltpu.async_copy(ts_hbm.at[pl.ds(r0, _SUB)],
                                  a_v.at[slot], sema)
            cb = pltpu.async_copy(te_hbm.at[pl.ds(r0, _SUB + _MAX_SPAN)],
                                  b_v.at[slot], semb)
            return ca, cb

        pend_in = issue_in(0)
        pltpu.sync_copy(wpar_hbm.at[0, pl.ds(0, _H)], w_v)
        pltpu.sync_copy(wpar_hbm.at[0, pl.ds(_H, 16)], bi_v)
        pend_out = [(), ()]
        for sub in range(_NSUBS):
            slot = sub % 2
            for c in pend_in:
                c.wait()
            if sub + 1 < _NSUBS:
                pend_in = issue_in(sub + 1)
            a_s = a_v.at[slot]
            b_s = b_v.at[slot]
            e_s = et_v.at[slot]
            c_s = sc_v.at[slot]
            for c in pend_out[slot]:
                c.wait()
            pend_out[slot] = ()
            s0b = s0t + sub * _SUB

            def pair_body(p, carry2, a_s=a_s, b_s=b_s, e_s=e_s, c_s=c_s,
                          s0b=s0b):
                i0 = p * 2

                def hb_body(t, accs):
                    for u in range(_HB_UNROLL):
                        hb = t * _HB_UNROLL + u
                        sl = pl.ds(hb * 16, 16)
                        va0 = a_s[i0, sl]
                        va1 = a_s[i0 + 1, sl]
                        vw = w_v[sl]
                        vbs = [b_s[i0 + k, sl] for k in range(_MAX_SPAN + 1)]
                        accs = tuple(
                            [accs[k]
                             + jnp.maximum(va0 + vbs[k], 0.0) * vw
                             for k in range(_MAX_SPAN)]
                            + [accs[_MAX_SPAN + k]
                               + jnp.maximum(va1 + vbs[k + 1], 0.0) * vw
                               for k in range(_MAX_SPAN)]
                        )
                    return accs

                accs = lax.fori_loop(0, _H // 16 // _HB_UNROLL, hb_body,
                                     (zero16,) * (2 * _MAX_SPAN))
                for j in range(2):
                    i = i0 + j
                    s = s0b + i
                    d = jnp.maximum(s - 505, 0)
                    lbase = 8 * i - (d * (d + 1)) // 2
                    lo, hi = pl.ds(_H, 16), pl.ds(_H + 16, 16)
                    ealo = a_s[i, lo]
                    eahi = a_s[i, hi]
                    xs = []
                    for k in range(_MAX_SPAN):
                        x = accs[j * _MAX_SPAN + k]
                        for sh in (8, 4, 2, 1):
                            x = x + jnp.take_along_axis(
                                x, lane ^ sh, axis=0,
                                mode="promise_in_bounds")
                        xs.append(x + bi_v[...])
                        lrow = jnp.full((16,), lbase + k, jnp.int32)
                        plsc.store_scatter(e_s, [lane, lrow],
                                           ealo + b_s[i + k, lo])
                        plsc.store_scatter(e_s, [lane + 16, lrow],
                                           eahi + b_s[i + k, hi])
                    svec = xs[7]
                    for k in range(7):
                        svec = jnp.where(lane == k, xs[k], svec)
                    plsc.store_scatter(c_s, [lbase + lane7], svec)
                return carry2

            lax.fori_loop(0, _SUB // 2, pair_body, 0)
            cbase = 8 * s0b
            co1 = pltpu.async_copy(
                c_s, osc_hbm.at[pl.ds(bb * _SEG + cbase, _BLK)], semo)
            co2 = pltpu.async_copy(
                et_v.at[slot, pl.ds(0, _NTP), pl.ds(0, _BLK)],
                oent_hbm.at[bb, :, pl.ds(cbase, _BLK)], semo)
            pend_out[slot] = (co1, co2)
        for pend in pend_out:
            for c in pend:
                c.wait()

    return span_kernel


def kernel(hidden_states, attention_mask, W_b, b_b, W_e, b_e,
           W_s1, b_s1, W_s2, b_s2):
    del attention_mask

    ts, te, bnd, wpar = _proj_call(
        hidden_states, W_s1, W_e, W_b, W_s2,
        b_s1.reshape(1, _H), b_e.reshape(1, _NT), b_b.reshape(1, 3),
        b_s2.reshape(1, 1))

    osc, oent = _span_call()(ts, te, wpar)

    span_scores = osc.reshape(_B, _SEG)[:, :_NSP].reshape(_B, _NSP, 1)
    entity_logits = oent[:, :_NT, :_NSP].transpose(0, 2, 1)
    return bnd, span_scores, entity_logits
